# baseline (speedup 1.0000x reference)
import jax
import jax.numpy as jnp
from jax.experimental import pallas as pl
from jax.experimental.pallas import tpu as pltpu

_B = 256
_V = 50257
_U = 65536
_M = 3

_CH = 1024
_VP = ((_V + _CH - 1) // _CH) * _CH
_CPM = _VP // _CH
_NCH = _M * _CPM
_HALF = _U // 2
_GRP = 8
_PIECE = 2048
_NFLUSH = _U // _PIECE


def _stats_body(x_ref, mx_ref, se_ref):
    c = pl.program_id(1)
    x = x_ref[0]

    @pl.when(c == 0)
    def _init():
        mx_ref[...] = jnp.full_like(mx_ref, -jnp.inf)
        se_ref[...] = jnp.zeros_like(se_ref)

    bm = jnp.max(x, axis=0, keepdims=True)
    m_old = mx_ref[0]
    m_new = jnp.maximum(m_old, bm)
    bs = jnp.sum(jnp.exp(x - m_new), axis=0, keepdims=True)
    se_ref[0] = se_ref[0] * jnp.exp(m_old - m_new) + bs
    mx_ref[0] = m_new


def _exp_body(w_ref, mx_ref, se_ref, x_ref, o_ref):
    x = x_ref[0]
    scale = w_ref[0] / se_ref[0]
    p = jnp.exp(x - mx_ref[0]) * scale
    o_ref[0, 0] = p[:, :128]
    o_ref[1, 0] = p[:, 128:]


def _scatter_body(cmap_ref, pt_ref, o_ref, acc, idx_smem, sem):
    c = pl.program_id(1)

    @pl.when(c < _NCH)
    def _accumulate():
        cp = pltpu.make_async_copy(cmap_ref.at[jnp.minimum(c, _NCH - 1)],
                                   idx_smem, sem)
        cp.start()

        @pl.when(c == 0)
        def _init():
            acc[...] = jnp.zeros_like(acc)

        cp.wait()

        def group(g, carry):
            rbase = g * _GRP
            for q in range(_GRP // 4):
                rb = rbase + q * 4
                a = [idx_smem[0, rb + i] for i in range(4)]
                v = [pt_ref[0, 0, rb + i] for i in range(4)]
                cur = [acc[a[i]] for i in range(4)]
                s = []
                for i in range(4):
                    si = cur[i] + v[i]
                    for j in range(i):
                        mij = jnp.where(a[i] == a[j],
                                        jnp.float32(1.0), jnp.float32(0.0))
                        si = si + v[j] * mij
                    s.append(si)
                for i in range(4):
                    acc[a[i]] = s[i]
            return carry

        jax.lax.fori_loop(0, _CH // _GRP, group, 0)

    @pl.when(c >= _NCH)
    def _flush():
        piece = c - _NCH
        o_ref[0] = acc[pl.ds(piece * _PIECE, _PIECE), 0, :]


def kernel(logits0, logits1, logits2, map0, map1, map2, weights):
    neg_inf = float("-inf")
    pad_w = _VP - _V
    lg = jnp.stack([
        jnp.pad(logits0, ((0, 0), (0, pad_w)), constant_values=neg_inf),
        jnp.pad(logits1, ((0, 0), (0, pad_w)), constant_values=neg_inf),
        jnp.pad(logits2, ((0, 0), (0, pad_w)), constant_values=neg_inf),
    ])
    lgt = jnp.transpose(lg, (0, 2, 1))
    w2 = jnp.broadcast_to(
        weights.astype(jnp.float32).reshape(_M, 1, 1), (_M, 1, _B))
    cmap = jnp.concatenate([
        jnp.pad(map0, (0, pad_w), constant_values=0),
        jnp.pad(map1, (0, pad_w), constant_values=0),
        jnp.pad(map2, (0, pad_w), constant_values=0),
    ]).reshape(_NCH, 1, _CH)

    mx, se = pl.pallas_call(
        _stats_body,
        grid=(_M, _CPM),
        in_specs=[pl.BlockSpec((1, _CH, _B), lambda m, c: (m, c, 0))],
        out_specs=[
            pl.BlockSpec((1, 1, _B), lambda m, c: (m, 0, 0)),
            pl.BlockSpec((1, 1, _B), lambda m, c: (m, 0, 0)),
        ],
        out_shape=[
            jax.ShapeDtypeStruct((_M, 1, _B), jnp.float32),
            jax.ShapeDtypeStruct((_M, 1, _B), jnp.float32),
        ],
        compiler_params=pltpu.CompilerParams(
            dimension_semantics=("parallel", "arbitrary")),
        name="ens_stats",
    )(lgt)

    pt = pl.pallas_call(
        _exp_body,
        grid=(_M, _CPM),
        in_specs=[
            pl.BlockSpec((1, 1, _B), lambda m, c: (m, 0, 0)),
            pl.BlockSpec((1, 1, _B), lambda m, c: (m, 0, 0)),
            pl.BlockSpec((1, 1, _B), lambda m, c: (m, 0, 0)),
            pl.BlockSpec((1, _CH, _B), lambda m, c: (m, c, 0)),
        ],
        out_specs=pl.BlockSpec((2, 1, _CH, 128), lambda m, c: (0, m, c, 0)),
        out_shape=jax.ShapeDtypeStruct((2, _M, _VP, 128), jnp.float32),
        compiler_params=pltpu.CompilerParams(
            dimension_semantics=("parallel", "arbitrary")),
        name="ens_exp",
    )(w2, mx, se, lgt)

    uniont = pl.pallas_call(
        _scatter_body,
        grid=(2, _NCH + _NFLUSH),
        in_specs=[
            pl.BlockSpec(memory_space=pltpu.VMEM),
            pl.BlockSpec((1, 1, _CH, 128),
                         lambda h, c: (h, jnp.minimum(c, _NCH - 1) // _CPM,
                                       jnp.minimum(c, _NCH - 1) % _CPM, 0)),
        ],
        out_specs=pl.BlockSpec((1, _PIECE, 128),
                               lambda h, c: (h, jnp.maximum(c - _NCH, 0), 0)),
        out_shape=jax.ShapeDtypeStruct((2, _U, 128), jnp.float32),
        scratch_shapes=[
            pltpu.VMEM((_U, 1, 128), jnp.float32),
            pltpu.SMEM((1, _CH), jnp.int32),
            pltpu.SemaphoreType.DMA,
        ],
        compiler_params=pltpu.CompilerParams(
            dimension_semantics=("parallel", "arbitrary")),
        name="ens_scatter",
    )(cmap, pt)

    return jnp.transpose(uniont, (0, 2, 1)).reshape(_B, _U)


# speedup vs baseline: 1.0739x; 1.0739x over previous
import jax
import jax.numpy as jnp
from jax.experimental import pallas as pl
from jax.experimental.pallas import tpu as pltpu

_B = 256
_V = 50257
_U = 65536
_M = 3

_CH = 1024
_VP = ((_V + _CH - 1) // _CH) * _CH
_CPM = _VP // _CH
_NCH = _M * _CPM
_HALF = _U // 2
_GRP = 8
_PIECE = 2048
_NFLUSH = _U // _PIECE


def _stats_body(x_ref, mx_ref, se_ref):
    c = pl.program_id(1)
    x = x_ref[0]

    @pl.when(c == 0)
    def _init():
        mx_ref[...] = jnp.full_like(mx_ref, -jnp.inf)
        se_ref[...] = jnp.zeros_like(se_ref)

    bm = jnp.max(x, axis=0, keepdims=True)
    m_old = mx_ref[0]
    m_new = jnp.maximum(m_old, bm)
    bs = jnp.sum(jnp.exp(x - m_new), axis=0, keepdims=True)
    se_ref[0] = se_ref[0] * jnp.exp(m_old - m_new) + bs
    mx_ref[0] = m_new


def _exp_body(w_ref, mx_ref, se_ref, x_ref, o_ref):
    x = x_ref[0]
    scale = w_ref[0] / se_ref[0]
    p = jnp.exp(x - mx_ref[0]) * scale
    o_ref[0, 0] = p[:, :128]
    o_ref[1, 0] = p[:, 128:]


def _scatter_body(cmap_ref, pt_ref, o_ref, acc, idx_smem, sem):
    c = pl.program_id(1)

    @pl.when(c < _NCH)
    def _accumulate():
        cp = pltpu.make_async_copy(cmap_ref.at[jnp.minimum(c, _NCH - 1)],
                                   idx_smem, sem)
        cp.start()

        @pl.when(c == 0)
        def _init():
            acc[...] = jnp.zeros_like(acc)

        cp.wait()

        def group(g, carry):
            rbase = g * _GRP
            for i in range(_GRP):
                r = rbase + i
                a = idx_smem[0, r]
                acc[a] = acc[a] + pt_ref[0, 0, r]
            return carry

        jax.lax.fori_loop(0, _CH // _GRP, group, 0)

    @pl.when(c >= _NCH)
    def _flush():
        piece = c - _NCH
        o_ref[...] = acc[pl.ds(piece * _PIECE, _PIECE), 0, :].T


def kernel(logits0, logits1, logits2, map0, map1, map2, weights):
    neg_inf = float("-inf")
    pad_w = _VP - _V
    lg = jnp.stack([
        jnp.pad(logits0, ((0, 0), (0, pad_w)), constant_values=neg_inf),
        jnp.pad(logits1, ((0, 0), (0, pad_w)), constant_values=neg_inf),
        jnp.pad(logits2, ((0, 0), (0, pad_w)), constant_values=neg_inf),
    ])
    lgt = jnp.transpose(lg, (0, 2, 1))
    w2 = jnp.broadcast_to(
        weights.astype(jnp.float32).reshape(_M, 1, 1), (_M, 1, _B))
    cmap = jnp.concatenate([
        jnp.pad(map0, (0, pad_w), constant_values=0),
        jnp.pad(map1, (0, pad_w), constant_values=0),
        jnp.pad(map2, (0, pad_w), constant_values=0),
    ]).reshape(_NCH, 1, _CH)

    mx, se = pl.pallas_call(
        _stats_body,
        grid=(_M, _CPM),
        in_specs=[pl.BlockSpec((1, _CH, _B), lambda m, c: (m, c, 0))],
        out_specs=[
            pl.BlockSpec((1, 1, _B), lambda m, c: (m, 0, 0)),
            pl.BlockSpec((1, 1, _B), lambda m, c: (m, 0, 0)),
        ],
        out_shape=[
            jax.ShapeDtypeStruct((_M, 1, _B), jnp.float32),
            jax.ShapeDtypeStruct((_M, 1, _B), jnp.float32),
        ],
        compiler_params=pltpu.CompilerParams(
            dimension_semantics=("parallel", "arbitrary")),
        name="ens_stats",
    )(lgt)

    pt = pl.pallas_call(
        _exp_body,
        grid=(_M, _CPM),
        in_specs=[
            pl.BlockSpec((1, 1, _B), lambda m, c: (m, 0, 0)),
            pl.BlockSpec((1, 1, _B), lambda m, c: (m, 0, 0)),
            pl.BlockSpec((1, 1, _B), lambda m, c: (m, 0, 0)),
            pl.BlockSpec((1, _CH, _B), lambda m, c: (m, c, 0)),
        ],
        out_specs=pl.BlockSpec((2, 1, _CH, 128), lambda m, c: (0, m, c, 0)),
        out_shape=jax.ShapeDtypeStruct((2, _M, _VP, 128), jnp.float32),
        compiler_params=pltpu.CompilerParams(
            dimension_semantics=("parallel", "arbitrary")),
        name="ens_exp",
    )(w2, mx, se, lgt)

    uniont = pl.pallas_call(
        _scatter_body,
        grid=(2, _NCH + _NFLUSH),
        in_specs=[
            pl.BlockSpec(memory_space=pltpu.VMEM),
            pl.BlockSpec((1, 1, _CH, 128),
                         lambda h, c: (h, jnp.minimum(c, _NCH - 1) // _CPM,
                                       jnp.minimum(c, _NCH - 1) % _CPM, 0)),
        ],
        out_specs=pl.BlockSpec((128, _PIECE),
                               lambda h, c: (h, jnp.maximum(c - _NCH, 0))),
        out_shape=jax.ShapeDtypeStruct((_B, _U), jnp.float32),
        scratch_shapes=[
            pltpu.VMEM((_U, 1, 128), jnp.float32),
            pltpu.SMEM((1, _CH), jnp.int32),
            pltpu.SemaphoreType.DMA,
        ],
        compiler_params=pltpu.CompilerParams(
            dimension_semantics=("parallel", "arbitrary")),
        name="ens_scatter",
    )(cmap, pt)

    return uniont
